# Initial kernel scaffold; baseline (speedup 1.0000x reference)
import jax
import jax.numpy as jnp
from jax.experimental import pallas as pl


def kernel(hidden_states, mask, enc_Wq, enc_bq, enc_Wk, enc_bk, dec_Wq, dec_bq, dec_Wk, dec_bk, residual_weights):
    raise NotImplementedError("write your pallas kernel here")


# speedup vs baseline: 1518.8631x; 1518.8631x over previous
import jax
import jax.numpy as jnp
from jax.experimental import pallas as pl

_L = 2048
_D = 256
_C = 256
_NCH = _L // _C


def _rms(x):
    return x * jax.lax.rsqrt(jnp.mean(x * x, axis=-1, keepdims=True) + 1e-6)


def _lane_iota():
    return jax.lax.broadcasted_iota(jnp.int32, (1, _L), 1).astype(jnp.float32)


def _eye_c():
    r = jax.lax.broadcasted_iota(jnp.int32, (_C, _C), 0)
    c = jax.lax.broadcasted_iota(jnp.int32, (_C, _C), 1)
    return r == c


def _col_to_row(col):
    eye = _eye_c()
    parts = []
    for c in range(_NCH):
        blk = jax.lax.slice(col, (c * _C, 0), ((c + 1) * _C, 1))
        m = jnp.where(eye, jnp.broadcast_to(blk, (_C, _C)), 0.0)
        parts.append(jnp.sum(m, axis=0, keepdims=True))
    return jnp.concatenate(parts, axis=1)


def _row_to_col_chunk(row):
    m = jnp.where(_eye_c(), jnp.broadcast_to(row, (_C, _C)), 0.0)
    return jnp.sum(m, axis=1, keepdims=True)


def _cumsum_row(x):
    s = x
    k = 1
    while k < _L:
        sh = jnp.concatenate(
            [jnp.zeros((1, k), x.dtype), jax.lax.slice(s, (0, 0), (1, _L - k))],
            axis=1)
        s = s + sh
        k *= 2
    return s


def _routing(x, m_row, Wq, bq, Wk, bk):
    dn = (((1,), (1,)), ((), ()))
    Qf = jax.lax.dot_general(x, Wq, dn, preferred_element_type=jnp.float32) + bq
    Kf = jax.lax.dot_general(x, Wk, dn, preferred_element_type=jnp.float32) + bk
    Ks = jnp.concatenate(
        [jax.lax.slice(Kf, (1, 0), (_L, _D)), jnp.zeros((1, _D), jnp.float32)],
        axis=0)
    qn = jnp.sqrt(jnp.sum(Qf * Qf, axis=1, keepdims=True))
    kn = jnp.sqrt(jnp.sum(Ks * Ks, axis=1, keepdims=True))
    s = jnp.sum(Qf * Ks, axis=1, keepdims=True)
    cos = s / (jnp.maximum(qn, 1e-12) * jnp.maximum(kn, 1e-12))
    a = jnp.clip(0.5 * (1.0 - cos), 0.0, 1.0)
    a_col = jnp.concatenate(
        [jnp.ones((1, 1), jnp.float32), jax.lax.slice(a, (0, 0), (_L - 1, 1))],
        axis=0)
    A_row = _col_to_row(a_col)
    bm_row = (A_row > 0.5) & m_row
    return A_row, bm_row


def _compact(e, bm_row):
    cum = _cumsum_row(bm_row.astype(jnp.float32))
    num = jax.lax.slice(cum, (0, _L - 1), (1, _L))
    dest = jnp.where(bm_row, cum - 1.0, num + _lane_iota() - cum)
    dn = (((1,), (0,)), ((), ()))
    rows = []
    for c in range(_NCH):
        ridx = (jax.lax.broadcasted_iota(jnp.int32, (_C, 1), 0).astype(jnp.float32)
                + float(c * _C))
        P = jnp.where(ridx == dest, 1.0, 0.0)
        rows.append(jax.lax.dot_general(P, e, dn, preferred_element_type=jnp.float32))
    return jnp.concatenate(rows, axis=0), num


def _upsample2(z1, z2, A_row, bm_row):
    z = jnp.concatenate([z1, z2], axis=1)
    p = jnp.clip(A_row, 1e-4, 1.0 - 1e-4)
    cum = _cumsum_row(bm_row.astype(jnp.float32))
    cb = jnp.maximum(cum - 1.0, 0.0)
    S = _cumsum_row(jnp.log(1.0 - p))
    Spad = jnp.concatenate(
        [jnp.zeros((1, 1), jnp.float32), jax.lax.slice(S, (0, 0), (1, _L - 1))],
        axis=1)
    r = jax.lax.broadcasted_iota(jnp.int32, (_C, _C), 0)
    c = jax.lax.broadcasted_iota(jnp.int32, (_C, _C), 1)
    tril = r >= c
    dn = (((1,), (0,)), ((), ()))
    lane = _lane_iota()
    carry = jnp.zeros((1, 2 * _D), jnp.float32)
    outs = []
    for ch in range(_NCH):
        s0 = ch * _C
        Sp_row = jax.lax.slice(Spad, (0, s0), (1, s0 + _C))
        p_row = jax.lax.slice(p, (0, s0), (1, s0 + _C))
        cb_col = _row_to_col_chunk(jax.lax.slice(cb, (0, s0), (1, s0 + _C)))
        Sp_col = _row_to_col_chunk(Sp_row)
        G = jnp.where(cb_col == lane, 1.0, 0.0)
        z_exp = jax.lax.dot_general(G, z, dn, preferred_element_type=jnp.float32)
        Wl = jnp.where(tril, p_row * jnp.exp(Sp_col - Sp_row), 0.0)
        loc = jax.lax.dot_general(Wl, z_exp, dn, preferred_element_type=jnp.float32)
        S0 = jax.lax.slice(Sp_row, (0, 0), (1, 1))
        out_c = loc + jnp.exp(Sp_col - S0) * carry
        p_last = jax.lax.slice(p_row, (0, _C - 1), (1, _C))
        out_last = jax.lax.slice(out_c, (_C - 1, 0), (_C, 2 * _D))
        carry = (1.0 - p_last) * out_last
        outs.append(out_c)
    return jnp.concatenate(outs, axis=0)


def _body(h_ref, mask_ref, eWq_ref, ebq_ref, eWk_ref, ebk_ref,
          dWq_ref, dbq_ref, dWk_ref, dbk_ref, rw_ref, out_ref):
    h0 = h_ref[0]
    m0 = mask_ref[0] > 0.5
    lane = _lane_iota()

    e0 = _rms(h0)
    _, bm0 = _routing(e0, m0, eWq_ref[0], ebq_ref[0], eWk_ref[0], ebk_ref[0])
    h1, num1 = _compact(e0, bm0)
    m1 = lane < num1

    e1 = _rms(h1)
    _, bm1 = _routing(e1, m1, eWq_ref[1], ebq_ref[1], eWk_ref[1], ebk_ref[1])
    h2, num2 = _compact(e1, bm1)
    m2 = lane < num2

    rw = rw_ref[0]
    d0 = _rms(h2)
    A0, bmA = _routing(d0, m2, dWq_ref[0], dbq_ref[0], dWk_ref[0], dbk_ref[0])
    u = _upsample2(d0, e1, A0, bmA)
    h3 = (jax.lax.slice(u, (0, 0), (_L, _D))
          + jax.lax.slice(rw, (0, 0), (1, 1))
          * jax.lax.slice(u, (0, _D), (_L, 2 * _D)))

    d1 = _rms(h3)
    A1, bmB = _routing(d1, m2, dWq_ref[1], dbq_ref[1], dWk_ref[1], dbk_ref[1])
    u2 = _upsample2(d1, e0, A1, bmB)
    out_ref[0] = (jax.lax.slice(u2, (0, 0), (_L, _D))
                  + jax.lax.slice(rw, (0, 1), (1, 2))
                  * jax.lax.slice(u2, (0, _D), (_L, 2 * _D)))


def kernel(hidden_states, mask, enc_Wq, enc_bq, enc_Wk, enc_bk,
           dec_Wq, dec_bq, dec_Wk, dec_bk, residual_weights):
    B = hidden_states.shape[0]
    nl = enc_Wq.shape[0]
    mask_f = mask.astype(jnp.float32).reshape(B, 1, _L)
    ebq = enc_bq.reshape(nl, 1, _D)
    ebk = enc_bk.reshape(nl, 1, _D)
    dbq = dec_bq.reshape(nl, 1, _D)
    dbk = dec_bk.reshape(nl, 1, _D)
    rw = residual_weights.astype(jnp.float32).reshape(1, 1, nl)

    full = lambda shape: pl.BlockSpec(shape, lambda b: (0,) * len(shape))
    out = pl.pallas_call(
        _body,
        grid=(B,),
        in_specs=[
            pl.BlockSpec((1, _L, _D), lambda b: (b, 0, 0)),
            pl.BlockSpec((1, 1, _L), lambda b: (b, 0, 0)),
            full((nl, _D, _D)), full((nl, 1, _D)),
            full((nl, _D, _D)), full((nl, 1, _D)),
            full((nl, _D, _D)), full((nl, 1, _D)),
            full((nl, _D, _D)), full((nl, 1, _D)),
            full((1, 1, nl)),
        ],
        out_specs=pl.BlockSpec((1, _L, _D), lambda b: (b, 0, 0)),
        out_shape=jax.ShapeDtypeStruct((B, _L, _D), jnp.float32),
    )(hidden_states, mask_f, enc_Wq, ebq, enc_Wk, ebk,
      dec_Wq, dbq, dec_Wk, dbk, rw)
    return out


# speedup vs baseline: 1575.5746x; 1.0373x over previous
import jax
import jax.numpy as jnp
from jax.experimental import pallas as pl
from jax.experimental.pallas import tpu as pltpu

_L = 2048
_D = 256
_C = 256
_NCH = _L // _C


def _rms(x):
    return x * jax.lax.rsqrt(jnp.mean(x * x, axis=-1, keepdims=True) + 1e-6)


def _lane_iota():
    return jax.lax.broadcasted_iota(jnp.int32, (1, _L), 1).astype(jnp.float32)


def _eye_c():
    r = jax.lax.broadcasted_iota(jnp.int32, (_C, _C), 0)
    c = jax.lax.broadcasted_iota(jnp.int32, (_C, _C), 1)
    return r == c


def _col_to_row(col):
    eye = _eye_c()
    parts = []
    for c in range(_NCH):
        blk = jax.lax.slice(col, (c * _C, 0), ((c + 1) * _C, 1))
        m = jnp.where(eye, jnp.broadcast_to(blk, (_C, _C)), 0.0)
        parts.append(jnp.sum(m, axis=0, keepdims=True))
    return jnp.concatenate(parts, axis=1)


def _row_to_col_chunk(row):
    m = jnp.where(_eye_c(), jnp.broadcast_to(row, (_C, _C)), 0.0)
    return jnp.sum(m, axis=1, keepdims=True)


def _cumsum_row(x):
    s = x
    k = 1
    while k < _L:
        sh = jnp.concatenate(
            [jnp.zeros((1, k), x.dtype), jax.lax.slice(s, (0, 0), (1, _L - k))],
            axis=1)
        s = s + sh
        k *= 2
    return s


def _routing(x, m_row, Wq, bq, Wk, bk):
    dn = (((1,), (1,)), ((), ()))
    Qf = jax.lax.dot_general(x, Wq, dn, preferred_element_type=jnp.float32) + bq
    Kf = jax.lax.dot_general(x, Wk, dn, preferred_element_type=jnp.float32) + bk
    Ks = jnp.concatenate(
        [jax.lax.slice(Kf, (1, 0), (_L, _D)), jnp.zeros((1, _D), jnp.float32)],
        axis=0)
    qn = jnp.sqrt(jnp.sum(Qf * Qf, axis=1, keepdims=True))
    kn = jnp.sqrt(jnp.sum(Ks * Ks, axis=1, keepdims=True))
    Qn = Qf / jnp.maximum(qn, 1e-12)
    Kn = Ks / jnp.maximum(kn, 1e-12)
    cos = jnp.sum(Qn * Kn, axis=1, keepdims=True)
    a = jnp.clip(0.5 * (1.0 - cos), 0.0, 1.0)
    a_col = jnp.concatenate(
        [jnp.ones((1, 1), jnp.float32), jax.lax.slice(a, (0, 0), (_L - 1, 1))],
        axis=0)
    A_row = _col_to_row(a_col)
    bm_row = (A_row > 0.5) & m_row
    return A_row, bm_row


def _compact(e, bm_row):
    cum = _cumsum_row(bm_row.astype(jnp.float32))
    num = jax.lax.slice(cum, (0, _L - 1), (1, _L))
    dest = jnp.where(bm_row, cum - 1.0, num + _lane_iota() - cum)
    dn = (((1,), (0,)), ((), ()))
    rows = []
    for c in range(_NCH):
        ridx = (jax.lax.broadcasted_iota(jnp.int32, (_C, 1), 0).astype(jnp.float32)
                + float(c * _C))
        P = jnp.where(ridx == dest, 1.0, 0.0)
        rows.append(jax.lax.dot_general(P, e, dn, preferred_element_type=jnp.float32))
    return jnp.concatenate(rows, axis=0), num


_W = 384


def _upsample2(z_ref, sc_ref, z1, z2, A_row, bm_row):
    z_ref[:, 0:_D] = z1
    z_ref[:, _D:2 * _D] = z2
    p = jnp.clip(A_row, 1e-4, 1.0 - 1e-4)
    cum = _cumsum_row(bm_row.astype(jnp.float32))
    cb = jnp.maximum(cum - 1.0, 0.0)
    sc_ref[0:1, :] = cb
    S = _cumsum_row(jnp.log(1.0 - p))
    Spad = jnp.concatenate(
        [jnp.zeros((1, 1), jnp.float32), jax.lax.slice(S, (0, 0), (1, _L - 1))],
        axis=1)
    r = jax.lax.broadcasted_iota(jnp.int32, (_C, _C), 0)
    c = jax.lax.broadcasted_iota(jnp.int32, (_C, _C), 1)
    tril = r >= c
    dn = (((1,), (0,)), ((), ()))
    lane_w = jax.lax.broadcasted_iota(jnp.int32, (1, _W), 1).astype(jnp.float32)
    carry = jnp.zeros((1, 2 * _D), jnp.float32)
    outs = []
    for ch in range(_NCH):
        s0 = ch * _C
        Sp_row = jax.lax.slice(Spad, (0, s0), (1, s0 + _C))
        p_row = jax.lax.slice(p, (0, s0), (1, s0 + _C))
        cb_col = _row_to_col_chunk(jax.lax.slice(cb, (0, s0), (1, s0 + _C)))
        Sp_col = _row_to_col_chunk(Sp_row)
        base = jnp.minimum((sc_ref[0, s0].astype(jnp.int32) // 128) * 128,
                           _L - _W)
        zwin = z_ref[pl.ds(base, _W), :]
        G = jnp.where(cb_col - base.astype(jnp.float32) == lane_w, 1.0, 0.0)
        z_exp = jax.lax.dot_general(G, zwin, dn, preferred_element_type=jnp.float32)
        Wl = jnp.where(tril, p_row * jnp.exp(Sp_col - Sp_row), 0.0)
        loc = jax.lax.dot_general(Wl, z_exp, dn, preferred_element_type=jnp.float32)
        S0 = jax.lax.slice(Sp_row, (0, 0), (1, 1))
        out_c = loc + jnp.exp(Sp_col - S0) * carry
        p_last = jax.lax.slice(p_row, (0, _C - 1), (1, _C))
        out_last = jax.lax.slice(out_c, (_C - 1, 0), (_C, 2 * _D))
        carry = (1.0 - p_last) * out_last
        outs.append(out_c)
    return jnp.concatenate(outs, axis=0)


def _body(h_ref, mask_ref, eWq_ref, ebq_ref, eWk_ref, ebk_ref,
          dWq_ref, dbq_ref, dWk_ref, dbk_ref, rw_ref, out_ref,
          z_ref, sc_ref):
    h0 = h_ref[0]
    m0 = mask_ref[0] > 0.5
    lane = _lane_iota()

    e0 = _rms(h0)
    _, bm0 = _routing(e0, m0, eWq_ref[0], ebq_ref[0], eWk_ref[0], ebk_ref[0])
    h1, num1 = _compact(e0, bm0)
    m1 = lane < num1

    e1 = _rms(h1)
    _, bm1 = _routing(e1, m1, eWq_ref[1], ebq_ref[1], eWk_ref[1], ebk_ref[1])
    h2, num2 = _compact(e1, bm1)
    m2 = lane < num2

    rw = rw_ref[0]
    d0 = _rms(h2)
    A0, bmA = _routing(d0, m2, dWq_ref[0], dbq_ref[0], dWk_ref[0], dbk_ref[0])
    u = _upsample2(z_ref, sc_ref, d0, e1, A0, bmA)
    h3 = (jax.lax.slice(u, (0, 0), (_L, _D))
          + jax.lax.slice(rw, (0, 0), (1, 1))
          * jax.lax.slice(u, (0, _D), (_L, 2 * _D)))

    d1 = _rms(h3)
    A1, bmB = _routing(d1, m2, dWq_ref[1], dbq_ref[1], dWk_ref[1], dbk_ref[1])
    u2 = _upsample2(z_ref, sc_ref, d1, e0, A1, bmB)
    out_ref[0] = (jax.lax.slice(u2, (0, 0), (_L, _D))
                  + jax.lax.slice(rw, (0, 1), (1, 2))
                  * jax.lax.slice(u2, (0, _D), (_L, 2 * _D)))


def kernel(hidden_states, mask, enc_Wq, enc_bq, enc_Wk, enc_bk,
           dec_Wq, dec_bq, dec_Wk, dec_bk, residual_weights):
    B = hidden_states.shape[0]
    nl = enc_Wq.shape[0]
    mask_f = mask.astype(jnp.float32).reshape(B, 1, _L)
    ebq = enc_bq.reshape(nl, 1, _D)
    ebk = enc_bk.reshape(nl, 1, _D)
    dbq = dec_bq.reshape(nl, 1, _D)
    dbk = dec_bk.reshape(nl, 1, _D)
    rw = residual_weights.astype(jnp.float32).reshape(1, 1, nl)

    full = lambda shape: pl.BlockSpec(shape, lambda b: (0,) * len(shape))
    out = pl.pallas_call(
        _body,
        grid=(B,),
        in_specs=[
            pl.BlockSpec((1, _L, _D), lambda b: (b, 0, 0)),
            pl.BlockSpec((1, 1, _L), lambda b: (b, 0, 0)),
            full((nl, _D, _D)), full((nl, 1, _D)),
            full((nl, _D, _D)), full((nl, 1, _D)),
            full((nl, _D, _D)), full((nl, 1, _D)),
            full((nl, _D, _D)), full((nl, 1, _D)),
            full((1, 1, nl)),
        ],
        out_specs=pl.BlockSpec((1, _L, _D), lambda b: (b, 0, 0)),
        out_shape=jax.ShapeDtypeStruct((B, _L, _D), jnp.float32),
        scratch_shapes=[pltpu.VMEM((_L, 2 * _D), jnp.float32),
                        pltpu.VMEM((1, _L), jnp.float32)],
    )(hidden_states, mask_f, enc_Wq, ebq, enc_Wk, ebk,
      dec_Wq, dbq, dec_Wk, dbk, rw)
    return out
